# Initial kernel scaffold; baseline (speedup 1.0000x reference)
import jax
import jax.numpy as jnp
from jax.experimental import pallas as pl


def kernel(input, gt_box, im_info, W_feat, b_feat, W_cls, b_cls, W_reg, b_reg):
    raise NotImplementedError("write your pallas kernel here")


# speedup vs baseline: 10.1716x; 10.1716x over previous
import numpy as np
import jax
import jax.numpy as jnp
from jax.experimental import pallas as pl
from jax.experimental.pallas import tpu as pltpu

_ASPECT = (0.5, 1.0, 2.0)
_SCALE = (8, 16, 32)
_STRIDE = 16
_PRE_NMS = 6000
_POST_NMS = 300
_NMS_THRESH = 0.7
_MIN_SIZE = 16.0
_H = 64
_W = 64
_A = 9
_N = _H * _W * _A
_ROWS, _LANES = 288, 128


def _np_anchor_fields():
    base_size = 16.0
    x_ctr = y_ctr = 0.5 * (base_size - 1.0)
    size = base_size * base_size
    base = []
    for r in _ASPECT:
        ws = np.round(np.sqrt(size / r))
        hs = np.round(ws * r)
        for s in _SCALE:
            wss, hss = ws * s, hs * s
            base.append([x_ctr - 0.5 * (wss - 1.0), y_ctr - 0.5 * (hss - 1.0),
                         x_ctr + 0.5 * (wss - 1.0), y_ctr + 0.5 * (hss - 1.0)])
    base = np.array(base, np.float32)
    sx = np.arange(_W, dtype=np.float32) * _STRIDE
    sy = np.arange(_H, dtype=np.float32) * _STRIDE
    yy, xx = np.meshgrid(sy, sx, indexing='ij')
    shifts = np.stack([xx, yy, xx, yy], -1).reshape(-1, 1, 4)
    anch = (shifts + base[None, :, :]).reshape(_N, 4).astype(np.float32)
    widths = anch[:, 2] - anch[:, 0] + 1.0
    heights = anch[:, 3] - anch[:, 1] + 1.0
    cx = anch[:, 0] + 0.5 * (widths - 1.0)
    cy = anch[:, 1] + 0.5 * (heights - 1.0)
    return [f.reshape(_ROWS, _LANES) for f in (widths, heights, cx, cy)]


_ANCHOR_FIELDS = _np_anchor_fields()


def _conv_body(f_ref, w_ref, bf_ref, wh_ref, bh_ref, out_ref, acc_ref):
    k = pl.program_id(0)

    @pl.when(k == 0)
    def _init():
        acc_ref[...] = jnp.zeros_like(acc_ref)

    acc_ref[...] += jnp.dot(f_ref[0], w_ref[0],
                            preferred_element_type=jnp.float32)

    @pl.when(k == 8)
    def _finish():
        feat = jnp.maximum(acc_ref[...] + bf_ref[0:1, :], 0.0)
        out_ref[...] = (jnp.dot(feat, wh_ref[...],
                                preferred_element_type=jnp.float32)
                        + bh_ref[0:1, :])


def _propose_body(info_ref, l0_ref, l1_ref, dx_ref, dy_ref, dw_ref, dh_ref,
                  aw_ref, ah_ref, ax_ref, ay_ref, out_ref):
    im_h = info_ref[0, 0]
    im_w = info_ref[0, 1]
    min_sz = _MIN_SIZE * info_ref[0, 2]

    fg = jax.nn.sigmoid(l1_ref[...] - l0_ref[...])
    wd = aw_ref[...]
    ht = ah_ref[...]
    cx = ax_ref[...]
    cy = ay_ref[...]
    dw = jnp.clip(dw_ref[...], -10.0, 4.135)
    dh = jnp.clip(dh_ref[...], -10.0, 4.135)
    pcx = dx_ref[...] * wd + cx
    pcy = dy_ref[...] * ht + cy
    pw = jnp.exp(dw) * wd
    ph = jnp.exp(dh) * ht
    x1 = jnp.clip(pcx - 0.5 * (pw - 1.0), 0.0, im_w - 1.0)
    y1 = jnp.clip(pcy - 0.5 * (ph - 1.0), 0.0, im_h - 1.0)
    x2 = jnp.clip(pcx + 0.5 * (pw - 1.0), 0.0, im_w - 1.0)
    y2 = jnp.clip(pcy + 0.5 * (ph - 1.0), 0.0, im_h - 1.0)
    ws = x2 - x1 + 1.0
    hs = y2 - y1 + 1.0
    s = jnp.where((ws >= min_sz) & (hs >= min_sz), fg, -1e9)
    areas = ws * hs

    def bis_body(_, lohi):
        lo, hi = lohi
        mid = lo + (hi - lo) * 0.5
        cnt = jnp.sum(jnp.where(s > mid, 1.0, 0.0))
        big = cnt >= _PRE_NMS
        return (jnp.where(big, mid, lo), jnp.where(big, hi, mid))

    lo, _ = jax.lax.fori_loop(
        0, 64, bis_body, (jnp.float32(-2e9), jnp.float32(2.0)))
    v = jnp.min(jnp.where(s > lo, s, jnp.inf))
    cnt_gt = jnp.sum(jnp.where(s > v, 1.0, 0.0))
    m = jnp.float32(_PRE_NMS) - cnt_gt

    tie = s == v
    tie_f = tie.astype(jnp.float32)
    ut = (jax.lax.broadcasted_iota(jnp.int32, (_LANES, _LANES), 0)
          <= jax.lax.broadcasted_iota(jnp.int32, (_LANES, _LANES), 1)
          ).astype(jnp.float32)
    lane_cum = jnp.dot(tie_f, ut, preferred_element_type=jnp.float32)
    row_tot = jnp.sum(tie_f, axis=1, keepdims=True)
    lt = (jax.lax.broadcasted_iota(jnp.int32, (_ROWS, _ROWS), 1)
          < jax.lax.broadcasted_iota(jnp.int32, (_ROWS, _ROWS), 0)
          ).astype(jnp.float32)
    row_pre = jnp.dot(lt, jnp.broadcast_to(row_tot, (_ROWS, _LANES)),
                      preferred_element_type=jnp.float32)
    rank = row_pre + lane_cum - tie_f
    in_top = (s > v) | (tie & (rank < m))

    a_flat = (jax.lax.broadcasted_iota(jnp.int32, (_ROWS, _LANES), 0) * _LANES
              + jax.lax.broadcasted_iota(jnp.int32, (_ROWS, _LANES), 1))
    neg_inf = jnp.float32(-jnp.inf)
    big_idx = jnp.int32(2**31 - 1)

    s_top0 = jnp.where(in_top, s, neg_inf)
    m0 = jnp.max(s_top0)
    idx0 = jnp.min(jnp.where(s_top0 == m0, a_flat, big_idx))
    oh0 = a_flat == idx0
    b0x1 = jnp.sum(jnp.where(oh0, x1, 0.0))
    b0y1 = jnp.sum(jnp.where(oh0, y1, 0.0))
    b0x2 = jnp.sum(jnp.where(oh0, x2, 0.0))
    b0y2 = jnp.sum(jnp.where(oh0, y2, 0.0))

    lane8 = jax.lax.broadcasted_iota(jnp.int32, (1, 8), 1)

    def nms_body(i, alive):
        masked = jnp.where(alive != 0.0, s, neg_inf)
        mx = jnp.max(masked)
        idx = jnp.min(jnp.where(masked == mx, a_flat, big_idx))
        oh = a_flat == idx
        ex = mx == neg_inf
        bx1 = jnp.where(ex, b0x1, jnp.sum(jnp.where(oh, x1, 0.0)))
        by1 = jnp.where(ex, b0y1, jnp.sum(jnp.where(oh, y1, 0.0)))
        bx2 = jnp.where(ex, b0x2, jnp.sum(jnp.where(oh, x2, 0.0)))
        by2 = jnp.where(ex, b0y2, jnp.sum(jnp.where(oh, y2, 0.0)))
        barea = (bx2 - bx1 + 1.0) * (by2 - by1 + 1.0)
        xx1 = jnp.maximum(bx1, x1)
        yy1 = jnp.maximum(by1, y1)
        xx2 = jnp.minimum(bx2, x2)
        yy2 = jnp.minimum(by2, y2)
        inter = (jnp.maximum(xx2 - xx1 + 1.0, 0.0)
                 * jnp.maximum(yy2 - yy1 + 1.0, 0.0))
        iou = inter / (barea + areas - inter)
        alive = jnp.where((iou <= _NMS_THRESH) & jnp.logical_not(oh),
                          alive, 0.0)
        row = (jnp.where(lane8 == 1, bx1, 0.0)
               + jnp.where(lane8 == 2, by1, 0.0)
               + jnp.where(lane8 == 3, bx2, 0.0)
               + jnp.where(lane8 == 4, by2, 0.0))
        out_ref[pl.ds(i, 1), :] = row
        return alive

    jax.lax.fori_loop(0, _POST_NMS, nms_body, in_top.astype(jnp.float32))


def kernel(input, gt_box, im_info, W_feat, b_feat, W_cls, b_cls, W_reg, b_reg):
    del gt_box
    xt = jnp.transpose(input[0], (1, 2, 0))
    xpad = jnp.pad(xt, ((1, 1), (1, 1), (0, 0)))
    f9 = jnp.stack([xpad[ky:ky + _H, kx:kx + _W, :].reshape(_H * _W, 256)
                    for ky in range(3) for kx in range(3)])
    w9 = jnp.transpose(W_feat, (2, 3, 1, 0)).reshape(9, 256, 512)
    bf2 = jnp.zeros((8, 512), jnp.float32).at[0].set(b_feat)
    wh = jnp.concatenate([W_cls[:, :, 0, 0].T, W_reg[:, :, 0, 0].T], axis=1)
    wh = jnp.pad(wh, ((0, 0), (0, 10)))
    bh2 = jnp.zeros((8, 64), jnp.float32).at[0].set(
        jnp.pad(jnp.concatenate([b_cls, b_reg]), (0, 10)))

    head = pl.pallas_call(
        _conv_body,
        grid=(9,),
        in_specs=[
            pl.BlockSpec((1, _H * _W, 256), lambda k: (k, 0, 0)),
            pl.BlockSpec((1, 256, 512), lambda k: (k, 0, 0)),
            pl.BlockSpec((8, 512), lambda k: (0, 0)),
            pl.BlockSpec((512, 64), lambda k: (0, 0)),
            pl.BlockSpec((8, 64), lambda k: (0, 0)),
        ],
        out_specs=pl.BlockSpec((_H * _W, 64), lambda k: (0, 0)),
        out_shape=jax.ShapeDtypeStruct((_H * _W, 64), jnp.float32),
        scratch_shapes=[pltpu.VMEM((_H * _W, 512), jnp.float32)],
    )(f9, w9, bf2, wh, bh2)

    head3 = head.reshape(_H, _W, 64)
    l0 = head3[:, :, 0:18:2].reshape(_ROWS, _LANES)
    l1 = head3[:, :, 1:18:2].reshape(_ROWS, _LANES)
    dx = head3[:, :, 18:54:4].reshape(_ROWS, _LANES)
    dy = head3[:, :, 19:54:4].reshape(_ROWS, _LANES)
    dw = head3[:, :, 20:54:4].reshape(_ROWS, _LANES)
    dh = head3[:, :, 21:54:4].reshape(_ROWS, _LANES)
    aw, ah, ax, ay = [jnp.asarray(f) for f in _ANCHOR_FIELDS]

    full = pl.BlockSpec((_ROWS, _LANES), lambda: (0, 0))
    res = pl.pallas_call(
        _propose_body,
        in_specs=[pl.BlockSpec(memory_space=pltpu.SMEM)] + [full] * 10,
        out_specs=pl.BlockSpec((_POST_NMS + 4, 8), lambda: (0, 0)),
        out_shape=jax.ShapeDtypeStruct((_POST_NMS + 4, 8), jnp.float32),
    )(im_info, l0, l1, dx, dy, dw, dh, aw, ah, ax, ay)

    return res[:_POST_NMS, :5]


# speedup vs baseline: 10.9531x; 1.0768x over previous
import numpy as np
import jax
import jax.numpy as jnp
from jax.experimental import pallas as pl
from jax.experimental.pallas import tpu as pltpu

_ASPECT = (0.5, 1.0, 2.0)
_SCALE = (8, 16, 32)
_STRIDE = 16
_PRE_NMS = 6000
_POST_NMS = 300
_NMS_THRESH = 0.7
_MIN_SIZE = 16.0
_H = 64
_W = 64
_A = 9
_N = _H * _W * _A
_ROWS, _LANES = 288, 128


def _np_anchor_fields():
    base_size = 16.0
    x_ctr = y_ctr = 0.5 * (base_size - 1.0)
    size = base_size * base_size
    base = []
    for r in _ASPECT:
        ws = np.round(np.sqrt(size / r))
        hs = np.round(ws * r)
        for s in _SCALE:
            wss, hss = ws * s, hs * s
            base.append([x_ctr - 0.5 * (wss - 1.0), y_ctr - 0.5 * (hss - 1.0),
                         x_ctr + 0.5 * (wss - 1.0), y_ctr + 0.5 * (hss - 1.0)])
    base = np.array(base, np.float32)
    sx = np.arange(_W, dtype=np.float32) * _STRIDE
    sy = np.arange(_H, dtype=np.float32) * _STRIDE
    yy, xx = np.meshgrid(sy, sx, indexing='ij')
    shifts = np.stack([xx, yy, xx, yy], -1).reshape(-1, 1, 4)
    anch = (shifts + base[None, :, :]).reshape(_N, 4).astype(np.float32)
    widths = anch[:, 2] - anch[:, 0] + 1.0
    heights = anch[:, 3] - anch[:, 1] + 1.0
    cx = anch[:, 0] + 0.5 * (widths - 1.0)
    cy = anch[:, 1] + 0.5 * (heights - 1.0)
    return [f.reshape(_ROWS, _LANES) for f in (widths, heights, cx, cy)]


_ANCHOR_FIELDS = _np_anchor_fields()


def _conv_body(f_ref, w_ref, bf_ref, wh_ref, bh_ref, out_ref, acc_ref):
    k = pl.program_id(0)

    @pl.when(k == 0)
    def _init():
        acc_ref[...] = jnp.zeros_like(acc_ref)

    acc_ref[...] += jnp.dot(f_ref[0], w_ref[0],
                            preferred_element_type=jnp.float32)

    @pl.when(k == 8)
    def _finish():
        feat = jnp.maximum(acc_ref[...] + bf_ref[0:1, :], 0.0)
        out_ref[...] = (jnp.dot(feat, wh_ref[...],
                                preferred_element_type=jnp.float32)
                        + bh_ref[0:1, :])


def _propose_body(info_ref, l0_ref, l1_ref, dx_ref, dy_ref, dw_ref, dh_ref,
                  aw_ref, ah_ref, ax_ref, ay_ref, out_ref,
                  sx1, sy1, sx2, sy2, sar):
    im_h = info_ref[0, 0]
    im_w = info_ref[0, 1]
    min_sz = _MIN_SIZE * info_ref[0, 2]

    fg = jax.nn.sigmoid(l1_ref[...] - l0_ref[...])
    wd = aw_ref[...]
    ht = ah_ref[...]
    cx = ax_ref[...]
    cy = ay_ref[...]
    dw = jnp.clip(dw_ref[...], -10.0, 4.135)
    dh = jnp.clip(dh_ref[...], -10.0, 4.135)
    pcx = dx_ref[...] * wd + cx
    pcy = dy_ref[...] * ht + cy
    pw = jnp.exp(dw) * wd
    ph = jnp.exp(dh) * ht
    x1 = jnp.clip(pcx - 0.5 * (pw - 1.0), 0.0, im_w - 1.0)
    y1 = jnp.clip(pcy - 0.5 * (ph - 1.0), 0.0, im_h - 1.0)
    x2 = jnp.clip(pcx + 0.5 * (pw - 1.0), 0.0, im_w - 1.0)
    y2 = jnp.clip(pcy + 0.5 * (ph - 1.0), 0.0, im_h - 1.0)
    ws = x2 - x1 + 1.0
    hs = y2 - y1 + 1.0
    s = jnp.where((ws >= min_sz) & (hs >= min_sz), fg, -1e9)
    areas = ws * hs
    sx1[...] = x1
    sy1[...] = y1
    sx2[...] = x2
    sy2[...] = y2
    sar[...] = areas

    def bis_body(_, lohi):
        lo, hi = lohi
        mid = lo + (hi - lo) * 0.5
        cnt = jnp.sum(jnp.where(s > mid, 1.0, 0.0))
        big = cnt >= _PRE_NMS
        return (jnp.where(big, mid, lo), jnp.where(big, hi, mid))

    c_nf = jnp.sum(jnp.where(s > -1.0, 1.0, 0.0))
    lo, _ = jax.lax.fori_loop(
        0, 64, bis_body, (jnp.float32(-1.0), jnp.float32(1.0)))
    v_bis = jnp.min(jnp.where(s > lo, s, jnp.inf))
    v = jnp.where(c_nf >= _PRE_NMS, v_bis, jnp.float32(-1e9))
    cnt_gt = jnp.sum(jnp.where(s > v, 1.0, 0.0))
    m = jnp.float32(_PRE_NMS) - cnt_gt

    tie = s == v
    tie_f = tie.astype(jnp.float32)
    ut = (jax.lax.broadcasted_iota(jnp.int32, (_LANES, _LANES), 0)
          <= jax.lax.broadcasted_iota(jnp.int32, (_LANES, _LANES), 1)
          ).astype(jnp.float32)
    lane_cum = jnp.dot(tie_f, ut, preferred_element_type=jnp.float32)
    row_tot = jnp.sum(tie_f, axis=1, keepdims=True)
    lt = (jax.lax.broadcasted_iota(jnp.int32, (_ROWS, _ROWS), 1)
          < jax.lax.broadcasted_iota(jnp.int32, (_ROWS, _ROWS), 0)
          ).astype(jnp.float32)
    row_pre = jnp.dot(lt, jnp.broadcast_to(row_tot, (_ROWS, _LANES)),
                      preferred_element_type=jnp.float32)
    rank = row_pre + lane_cum - tie_f
    in_top = (s > v) | (tie & (rank < m))

    a_flat = (jax.lax.broadcasted_iota(jnp.int32, (_ROWS, _LANES), 0) * _LANES
              + jax.lax.broadcasted_iota(jnp.int32, (_ROWS, _LANES), 1))
    neg_inf = jnp.float32(-jnp.inf)
    big_idx = jnp.int32(2**31 - 1)

    s_top0 = jnp.where(in_top, s, neg_inf)
    m0 = jnp.max(s_top0)
    idx0 = jnp.min(jnp.where(s_top0 == m0, a_flat, big_idx))
    oh0 = a_flat == idx0
    b0x1 = jnp.sum(jnp.where(oh0, x1, 0.0))
    b0y1 = jnp.sum(jnp.where(oh0, y1, 0.0))
    b0x2 = jnp.sum(jnp.where(oh0, x2, 0.0))
    b0y2 = jnp.sum(jnp.where(oh0, y2, 0.0))
    b0ar = jnp.sum(jnp.where(oh0, areas, 0.0))

    lane8 = jax.lax.broadcasted_iota(jnp.int32, (1, 8), 1)
    lane_iota = jax.lax.broadcasted_iota(jnp.int32, (1, _LANES), 1)

    def _pick(ref, r, lsel):
        return jnp.sum(jnp.where(lane_iota == lsel, ref[pl.ds(r, 1), :], 0.0))

    def nms_body(i, ms):
        mx = jnp.max(ms)
        idx = jnp.min(jnp.where(ms == mx, a_flat, big_idx))
        ex = mx == neg_inf
        r = jax.lax.select(ex, jnp.int32(0), idx // _LANES)
        lsel = jax.lax.select(ex, jnp.int32(0), idx % _LANES)
        bx1 = jnp.where(ex, b0x1, _pick(sx1, r, lsel))
        by1 = jnp.where(ex, b0y1, _pick(sy1, r, lsel))
        bx2 = jnp.where(ex, b0x2, _pick(sx2, r, lsel))
        by2 = jnp.where(ex, b0y2, _pick(sy2, r, lsel))
        barea = jnp.where(ex, b0ar, _pick(sar, r, lsel))
        xx1 = jnp.maximum(bx1, x1)
        yy1 = jnp.maximum(by1, y1)
        xx2 = jnp.minimum(bx2, x2)
        yy2 = jnp.minimum(by2, y2)
        inter = (jnp.maximum(xx2 - xx1 + 1.0, 0.0)
                 * jnp.maximum(yy2 - yy1 + 1.0, 0.0))
        iou = inter / (barea + areas - inter)
        ms = jnp.where(iou <= _NMS_THRESH, ms, neg_inf)
        row = (jnp.where(lane8 == 1, bx1, 0.0)
               + jnp.where(lane8 == 2, by1, 0.0)
               + jnp.where(lane8 == 3, bx2, 0.0)
               + jnp.where(lane8 == 4, by2, 0.0))
        out_ref[pl.ds(i, 1), :] = row
        return ms

    jax.lax.fori_loop(0, _POST_NMS, nms_body, s_top0)


def kernel(input, gt_box, im_info, W_feat, b_feat, W_cls, b_cls, W_reg, b_reg):
    del gt_box
    xt = jnp.transpose(input[0], (1, 2, 0))
    xpad = jnp.pad(xt, ((1, 1), (1, 1), (0, 0)))
    f9 = jnp.stack([xpad[ky:ky + _H, kx:kx + _W, :].reshape(_H * _W, 256)
                    for ky in range(3) for kx in range(3)])
    w9 = jnp.transpose(W_feat, (2, 3, 1, 0)).reshape(9, 256, 512)
    bf2 = jnp.zeros((8, 512), jnp.float32).at[0].set(b_feat)
    wh = jnp.concatenate([W_cls[:, :, 0, 0].T, W_reg[:, :, 0, 0].T], axis=1)
    wh = jnp.pad(wh, ((0, 0), (0, 10)))
    bh2 = jnp.zeros((8, 64), jnp.float32).at[0].set(
        jnp.pad(jnp.concatenate([b_cls, b_reg]), (0, 10)))

    head = pl.pallas_call(
        _conv_body,
        grid=(9,),
        in_specs=[
            pl.BlockSpec((1, _H * _W, 256), lambda k: (k, 0, 0)),
            pl.BlockSpec((1, 256, 512), lambda k: (k, 0, 0)),
            pl.BlockSpec((8, 512), lambda k: (0, 0)),
            pl.BlockSpec((512, 64), lambda k: (0, 0)),
            pl.BlockSpec((8, 64), lambda k: (0, 0)),
        ],
        out_specs=pl.BlockSpec((_H * _W, 64), lambda k: (0, 0)),
        out_shape=jax.ShapeDtypeStruct((_H * _W, 64), jnp.float32),
        scratch_shapes=[pltpu.VMEM((_H * _W, 512), jnp.float32)],
    )(f9, w9, bf2, wh, bh2)

    head3 = head.reshape(_H, _W, 64)
    l0 = head3[:, :, 0:18:2].reshape(_ROWS, _LANES)
    l1 = head3[:, :, 1:18:2].reshape(_ROWS, _LANES)
    dx = head3[:, :, 18:54:4].reshape(_ROWS, _LANES)
    dy = head3[:, :, 19:54:4].reshape(_ROWS, _LANES)
    dw = head3[:, :, 20:54:4].reshape(_ROWS, _LANES)
    dh = head3[:, :, 21:54:4].reshape(_ROWS, _LANES)
    aw, ah, ax, ay = [jnp.asarray(f) for f in _ANCHOR_FIELDS]

    full = pl.BlockSpec((_ROWS, _LANES), lambda: (0, 0))
    res = pl.pallas_call(
        _propose_body,
        in_specs=[pl.BlockSpec(memory_space=pltpu.SMEM)] + [full] * 10,
        out_specs=pl.BlockSpec((_POST_NMS + 4, 8), lambda: (0, 0)),
        out_shape=jax.ShapeDtypeStruct((_POST_NMS + 4, 8), jnp.float32),
        scratch_shapes=[pltpu.VMEM((_ROWS, _LANES), jnp.float32)] * 5,
    )(im_info, l0, l1, dx, dy, dw, dh, aw, ah, ax, ay)

    return res[:_POST_NMS, :5]


# speedup vs baseline: 11.2142x; 1.0238x over previous
import numpy as np
import jax
import jax.numpy as jnp
from jax.experimental import pallas as pl
from jax.experimental.pallas import tpu as pltpu

_ASPECT = (0.5, 1.0, 2.0)
_SCALE = (8, 16, 32)
_STRIDE = 16
_PRE_NMS = 6000
_POST_NMS = 300
_NMS_THRESH = 0.7
_MIN_SIZE = 16.0
_H = 64
_W = 64
_A = 9
_N = _H * _W * _A
_ROWS, _LANES = 288, 128


def _np_anchor_fields():
    base_size = 16.0
    x_ctr = y_ctr = 0.5 * (base_size - 1.0)
    size = base_size * base_size
    base = []
    for r in _ASPECT:
        ws = np.round(np.sqrt(size / r))
        hs = np.round(ws * r)
        for s in _SCALE:
            wss, hss = ws * s, hs * s
            base.append([x_ctr - 0.5 * (wss - 1.0), y_ctr - 0.5 * (hss - 1.0),
                         x_ctr + 0.5 * (wss - 1.0), y_ctr + 0.5 * (hss - 1.0)])
    base = np.array(base, np.float32)
    sx = np.arange(_W, dtype=np.float32) * _STRIDE
    sy = np.arange(_H, dtype=np.float32) * _STRIDE
    yy, xx = np.meshgrid(sy, sx, indexing='ij')
    shifts = np.stack([xx, yy, xx, yy], -1).reshape(-1, 1, 4)
    anch = (shifts + base[None, :, :]).reshape(_N, 4).astype(np.float32)
    widths = anch[:, 2] - anch[:, 0] + 1.0
    heights = anch[:, 3] - anch[:, 1] + 1.0
    cx = anch[:, 0] + 0.5 * (widths - 1.0)
    cy = anch[:, 1] + 0.5 * (heights - 1.0)
    return [f.reshape(_ROWS, _LANES) for f in (widths, heights, cx, cy)]


_ANCHOR_FIELDS = _np_anchor_fields()


def _conv_body(f_ref, w_ref, bf_ref, wh_ref, bh_ref, out_ref, acc_ref):
    k = pl.program_id(0)

    @pl.when(k == 0)
    def _init():
        acc_ref[...] = jnp.zeros_like(acc_ref)

    acc_ref[...] += jnp.dot(f_ref[0], w_ref[0],
                            preferred_element_type=jnp.float32)

    @pl.when(k == 8)
    def _finish():
        feat = jnp.maximum(acc_ref[...] + bf_ref[0:1, :], 0.0)
        out_ref[...] = (jnp.dot(feat, wh_ref[...],
                                preferred_element_type=jnp.float32)
                        + bh_ref[0:1, :])


def _propose_body(info_ref, l0_ref, l1_ref, dx_ref, dy_ref, dw_ref, dh_ref,
                  aw_ref, ah_ref, ax_ref, ay_ref, out_ref,
                  sx1, sy1, sx2, sy2, sar):
    im_h = info_ref[0, 0]
    im_w = info_ref[0, 1]
    min_sz = _MIN_SIZE * info_ref[0, 2]

    fg = jax.nn.sigmoid(l1_ref[...] - l0_ref[...])
    wd = aw_ref[...]
    ht = ah_ref[...]
    cx = ax_ref[...]
    cy = ay_ref[...]
    dw = jnp.clip(dw_ref[...], -10.0, 4.135)
    dh = jnp.clip(dh_ref[...], -10.0, 4.135)
    pcx = dx_ref[...] * wd + cx
    pcy = dy_ref[...] * ht + cy
    pw = jnp.exp(dw) * wd
    ph = jnp.exp(dh) * ht
    x1 = jnp.clip(pcx - 0.5 * (pw - 1.0), 0.0, im_w - 1.0)
    y1 = jnp.clip(pcy - 0.5 * (ph - 1.0), 0.0, im_h - 1.0)
    x2 = jnp.clip(pcx + 0.5 * (pw - 1.0), 0.0, im_w - 1.0)
    y2 = jnp.clip(pcy + 0.5 * (ph - 1.0), 0.0, im_h - 1.0)
    ws = x2 - x1 + 1.0
    hs = y2 - y1 + 1.0
    s = jnp.where((ws >= min_sz) & (hs >= min_sz), fg, -1e9)
    areas = ws * hs
    sx1[...] = x1
    sy1[...] = y1
    sx2[...] = x2
    sy2[...] = y2
    sar[...] = areas

    def bis_body(_, lohi):
        lo, hi = lohi
        mid = lo + (hi - lo) * 0.5
        cnt = jnp.sum(jnp.where(s > mid, 1.0, 0.0))
        big = cnt >= _PRE_NMS
        return (jnp.where(big, mid, lo), jnp.where(big, hi, mid))

    c_nf = jnp.sum(jnp.where(s > -1.0, 1.0, 0.0))
    lo, _ = jax.lax.fori_loop(
        0, 64, bis_body, (jnp.float32(-1.0), jnp.float32(1.0)))
    v_bis = jnp.min(jnp.where(s > lo, s, jnp.inf))
    v = jnp.where(c_nf >= _PRE_NMS, v_bis, jnp.float32(-1e9))
    cnt_gt = jnp.sum(jnp.where(s > v, 1.0, 0.0))
    m = jnp.float32(_PRE_NMS) - cnt_gt

    tie = s == v
    tie_f = tie.astype(jnp.float32)
    ut = (jax.lax.broadcasted_iota(jnp.int32, (_LANES, _LANES), 0)
          <= jax.lax.broadcasted_iota(jnp.int32, (_LANES, _LANES), 1)
          ).astype(jnp.float32)
    lane_cum = jnp.dot(tie_f, ut, preferred_element_type=jnp.float32)
    row_tot = jnp.sum(tie_f, axis=1, keepdims=True)
    lt = (jax.lax.broadcasted_iota(jnp.int32, (_ROWS, _ROWS), 1)
          < jax.lax.broadcasted_iota(jnp.int32, (_ROWS, _ROWS), 0)
          ).astype(jnp.float32)
    row_pre = jnp.dot(lt, jnp.broadcast_to(row_tot, (_ROWS, _LANES)),
                      preferred_element_type=jnp.float32)
    rank = row_pre + lane_cum - tie_f
    in_top = (s > v) | (tie & (rank < m))

    a_flat = (jax.lax.broadcasted_iota(jnp.int32, (_ROWS, _LANES), 0) * _LANES
              + jax.lax.broadcasted_iota(jnp.int32, (_ROWS, _LANES), 1))
    neg_inf = jnp.float32(-jnp.inf)
    big_idx = jnp.int32(2**31 - 1)

    s_top0 = jnp.where(in_top, s, neg_inf)
    m0 = jnp.max(s_top0)
    idx0 = jnp.min(jnp.where(s_top0 == m0, a_flat, big_idx))
    oh0 = a_flat == idx0
    b0x1 = jnp.sum(jnp.where(oh0, x1, 0.0))
    b0y1 = jnp.sum(jnp.where(oh0, y1, 0.0))
    b0x2 = jnp.sum(jnp.where(oh0, x2, 0.0))
    b0y2 = jnp.sum(jnp.where(oh0, y2, 0.0))
    b0ar = jnp.sum(jnp.where(oh0, areas, 0.0))

    lane8 = jax.lax.broadcasted_iota(jnp.int32, (1, 8), 1)
    lane_iota = jax.lax.broadcasted_iota(jnp.int32, (1, _LANES), 1)
    row_iota = jax.lax.broadcasted_iota(jnp.int32, (_ROWS, _LANES), 0)

    def _pick(ref, r, lsel):
        return jnp.sum(jnp.where(lane_iota == lsel, ref[pl.ds(r, 1), :], 0.0))

    def nms_body(i, ms):
        colmax = jnp.max(ms, axis=0, keepdims=True)
        rowhit = jnp.min(jnp.where(ms == colmax, row_iota, big_idx),
                         axis=0, keepdims=True)
        mx = jnp.max(colmax)
        idx = jnp.min(jnp.where(colmax == mx,
                                rowhit * _LANES + lane_iota, big_idx))
        ex = mx == neg_inf
        r = jax.lax.select(ex, jnp.int32(0), idx // _LANES)
        lsel = jax.lax.select(ex, jnp.int32(0), idx % _LANES)
        bx1 = jnp.where(ex, b0x1, _pick(sx1, r, lsel))
        by1 = jnp.where(ex, b0y1, _pick(sy1, r, lsel))
        bx2 = jnp.where(ex, b0x2, _pick(sx2, r, lsel))
        by2 = jnp.where(ex, b0y2, _pick(sy2, r, lsel))
        barea = jnp.where(ex, b0ar, _pick(sar, r, lsel))
        xx1 = jnp.maximum(bx1, x1)
        yy1 = jnp.maximum(by1, y1)
        xx2 = jnp.minimum(bx2, x2)
        yy2 = jnp.minimum(by2, y2)
        inter = (jnp.maximum(xx2 - xx1 + 1.0, 0.0)
                 * jnp.maximum(yy2 - yy1 + 1.0, 0.0))
        iou = inter / (barea + areas - inter)
        ms = jnp.where(iou <= _NMS_THRESH, ms, neg_inf)
        row = (jnp.where(lane8 == 1, bx1, 0.0)
               + jnp.where(lane8 == 2, by1, 0.0)
               + jnp.where(lane8 == 3, bx2, 0.0)
               + jnp.where(lane8 == 4, by2, 0.0))
        out_ref[pl.ds(i, 1), :] = row
        return ms

    jax.lax.fori_loop(0, _POST_NMS, nms_body, s_top0)


def kernel(input, gt_box, im_info, W_feat, b_feat, W_cls, b_cls, W_reg, b_reg):
    del gt_box
    xt = jnp.transpose(input[0], (1, 2, 0))
    xpad = jnp.pad(xt, ((1, 1), (1, 1), (0, 0)))
    f9 = jnp.stack([xpad[ky:ky + _H, kx:kx + _W, :].reshape(_H * _W, 256)
                    for ky in range(3) for kx in range(3)])
    w9 = jnp.transpose(W_feat, (2, 3, 1, 0)).reshape(9, 256, 512)
    bf2 = jnp.zeros((8, 512), jnp.float32).at[0].set(b_feat)
    wh = jnp.concatenate([W_cls[:, :, 0, 0].T, W_reg[:, :, 0, 0].T], axis=1)
    wh = jnp.pad(wh, ((0, 0), (0, 10)))
    bh2 = jnp.zeros((8, 64), jnp.float32).at[0].set(
        jnp.pad(jnp.concatenate([b_cls, b_reg]), (0, 10)))

    head = pl.pallas_call(
        _conv_body,
        grid=(9,),
        in_specs=[
            pl.BlockSpec((1, _H * _W, 256), lambda k: (k, 0, 0)),
            pl.BlockSpec((1, 256, 512), lambda k: (k, 0, 0)),
            pl.BlockSpec((8, 512), lambda k: (0, 0)),
            pl.BlockSpec((512, 64), lambda k: (0, 0)),
            pl.BlockSpec((8, 64), lambda k: (0, 0)),
        ],
        out_specs=pl.BlockSpec((_H * _W, 64), lambda k: (0, 0)),
        out_shape=jax.ShapeDtypeStruct((_H * _W, 64), jnp.float32),
        scratch_shapes=[pltpu.VMEM((_H * _W, 512), jnp.float32)],
    )(f9, w9, bf2, wh, bh2)

    head3 = head.reshape(_H, _W, 64)
    l0 = head3[:, :, 0:18:2].reshape(_ROWS, _LANES)
    l1 = head3[:, :, 1:18:2].reshape(_ROWS, _LANES)
    dx = head3[:, :, 18:54:4].reshape(_ROWS, _LANES)
    dy = head3[:, :, 19:54:4].reshape(_ROWS, _LANES)
    dw = head3[:, :, 20:54:4].reshape(_ROWS, _LANES)
    dh = head3[:, :, 21:54:4].reshape(_ROWS, _LANES)
    aw, ah, ax, ay = [jnp.asarray(f) for f in _ANCHOR_FIELDS]

    full = pl.BlockSpec((_ROWS, _LANES), lambda: (0, 0))
    res = pl.pallas_call(
        _propose_body,
        in_specs=[pl.BlockSpec(memory_space=pltpu.SMEM)] + [full] * 10,
        out_specs=pl.BlockSpec((_POST_NMS + 4, 8), lambda: (0, 0)),
        out_shape=jax.ShapeDtypeStruct((_POST_NMS + 4, 8), jnp.float32),
        scratch_shapes=[pltpu.VMEM((_ROWS, _LANES), jnp.float32)] * 5,
    )(im_info, l0, l1, dx, dy, dw, dh, aw, ah, ax, ay)

    return res[:_POST_NMS, :5]
